# Initial kernel scaffold; baseline (speedup 1.0000x reference)
import jax
import jax.numpy as jnp
from jax.experimental import pallas as pl


def kernel(x, edge_index, W1, b1, W2, b2):
    raise NotImplementedError("write your pallas kernel here")


# speedup vs baseline: 5.8272x; 5.8272x over previous
import functools

import jax
import jax.numpy as jnp
from jax import lax
from jax.experimental import pallas as pl
from jax.experimental.pallas import tpu as pltpu
from jax.experimental.pallas import tpu_sc as plsc

N = 10000
E = 320000
F_IN = 128
H = 64
C = 32
K = 10
ALPHA = 0.1

NC = 2
NS = 16
HC = C // 2
N_PAD = 10240
NPT = N_PAD // NS
EPT = E // NS
CHUNK = 80
NCHUNK = EPT // CHUNK


def _mlp_body(x_ref, w1_ref, b1_ref, w2_ref, b2_ref, o_ref):
    h1 = jnp.dot(x_ref[...], w1_ref[...], preferred_element_type=jnp.float32)
    h1 = jnp.maximum(h1 + b1_ref[...], 0.0)
    o_ref[...] = (
        jnp.dot(h1, w2_ref[...], preferred_element_type=jnp.float32)
        + b2_ref[...]
    )


def _mlp(x, W1, b1, W2, b2):
    blk = 1000
    return pl.pallas_call(
        _mlp_body,
        grid=(N // blk,),
        in_specs=[
            pl.BlockSpec((blk, F_IN), lambda i: (i, 0)),
            pl.BlockSpec((F_IN, H), lambda i: (0, 0)),
            pl.BlockSpec((1, H), lambda i: (0, 0)),
            pl.BlockSpec((H, C), lambda i: (0, 0)),
            pl.BlockSpec((1, C), lambda i: (0, 0)),
        ],
        out_specs=pl.BlockSpec((blk, C), lambda i: (i, 0)),
        out_shape=jax.ShapeDtypeStruct((N, C), jnp.float32),
    )(x, W1, b1.reshape(1, H), W2, b2.reshape(1, C))


def _lsm_body(z_ref, o_ref):
    z = z_ref[...]
    m = jnp.max(z, axis=1, keepdims=True)
    e = jnp.exp(z - m)
    s = jnp.sum(e, axis=1, keepdims=True)
    o_ref[...] = z - m - jnp.log(s)


def _log_softmax(z):
    blk = 1000
    return pl.pallas_call(
        _lsm_body,
        grid=(N // blk,),
        in_specs=[pl.BlockSpec((blk, C), lambda i: (i, 0))],
        out_specs=pl.BlockSpec((blk, C), lambda i: (i, 0)),
        out_shape=jax.ShapeDtypeStruct((N, C), jnp.float32),
    )(z)


def _prop_body(src_hbm, dst_hbm, h2_hbm, z2_hbm, zt_hbm,
               acc_sh, h_l, z_l, zt_l, dv_l, s_l, zeros_l,
               srci_v, idx_v, dsti_v, rows_v, ones_v, sem):
    c = lax.axis_index("c")
    s = lax.axis_index("s")
    nid0 = s * NPT
    g0 = c * N_PAD + nid0
    ebase = s * EPT
    cN = c * N_PAD

    pltpu.sync_copy(h2_hbm.at[pl.ds(g0, NPT)], h_l)

    def initrow(r, _):
        zeros_l[r, :] = jnp.zeros((16,), jnp.float32)
        return None
    lax.fori_loop(0, NPT, initrow, None)

    def onesrow(r, _):
        ones_v[r, :] = jnp.ones((16,), jnp.float32)
        return None
    lax.fori_loop(0, CHUNK, onesrow, None)

    pltpu.sync_copy(zeros_l, acc_sh.at[pl.ds(nid0, NPT)])
    plsc.subcore_barrier()

    def degchunk(ci, _):
        off = ebase + ci * CHUNK
        pltpu.sync_copy(dst_hbm.at[pl.ds(off, CHUNK)], dsti_v)
        pltpu.sync_copy(ones_v, acc_sh.at[dsti_v], add=True)
        return None
    lax.fori_loop(0, NCHUNK, degchunk, None)
    plsc.subcore_barrier()

    pltpu.sync_copy(acc_sh.at[pl.ds(nid0, NPT)], s_l)

    def dinvrow(r, _):
        x = s_l[r, :] + 1.0
        i = plsc.bitcast(x, jnp.int32)
        i = 0x5F3759DF - lax.shift_right_arithmetic(i, 1)
        y = plsc.bitcast(i, jnp.float32)
        y = y * (1.5 - 0.5 * x * y * y)
        y = y * (1.5 - 0.5 * x * y * y)
        y = y * (1.5 - 0.5 * x * y * y)
        dv_l[r, :] = y
        z_l[r, :] = h_l[r, :]
        return None
    lax.fori_loop(0, NPT, dinvrow, None)
    pltpu.sync_copy(zeros_l, acc_sh.at[pl.ds(nid0, NPT)])

    def k_iter(k, _):
        def ztrow(r, _):
            zt_l[r, :] = z_l[r, :] * dv_l[r, :]
            return None
        lax.fori_loop(0, NPT, ztrow, None)
        pltpu.sync_copy(zt_l, zt_hbm.at[pl.ds(g0, NPT)])
        plsc.subcore_barrier()

        def echunk(ci, _):
            off = ebase + ci * CHUNK
            pltpu.sync_copy(src_hbm.at[pl.ds(off, CHUNK)], srci_v)

            def addoff(j, _):
                idx_v[pl.ds(j * 16, 16)] = srci_v[pl.ds(j * 16, 16)] + cN
                return None
            lax.fori_loop(0, CHUNK // 16, addoff, None)
            pltpu.async_copy(zt_hbm.at[idx_v], rows_v, sem).wait()
            pltpu.sync_copy(dst_hbm.at[pl.ds(off, CHUNK)], dsti_v)
            pltpu.sync_copy(rows_v, acc_sh.at[dsti_v], add=True)
            return None
        lax.fori_loop(0, NCHUNK, echunk, None)
        plsc.subcore_barrier()

        pltpu.sync_copy(acc_sh.at[pl.ds(nid0, NPT)], s_l)
        pltpu.sync_copy(zeros_l, acc_sh.at[pl.ds(nid0, NPT)])

        def comb(r, _):
            dv = dv_l[r, :]
            agg = dv * s_l[r, :] + dv * dv * z_l[r, :]
            z_l[r, :] = (1.0 - ALPHA) * agg + ALPHA * h_l[r, :]
            return None
        lax.fori_loop(0, NPT, comb, None)
        return None
    lax.fori_loop(0, K, k_iter, None)

    pltpu.sync_copy(z_l, z2_hbm.at[pl.ds(g0, NPT)])


_propagate = functools.partial(
    pl.kernel,
    out_type=(
        jax.ShapeDtypeStruct((2 * N_PAD, HC), jnp.float32),
        jax.ShapeDtypeStruct((2 * N_PAD, HC), jnp.float32),
    ),
    mesh=plsc.VectorSubcoreMesh(core_axis_name="c", subcore_axis_name="s"),
    compiler_params=pltpu.CompilerParams(
        needs_layout_passes=False, use_tc_tiling_on_sc=False
    ),
    scratch_types=[
        pltpu.VMEM_SHARED((N_PAD, HC), jnp.float32),
        pltpu.VMEM((NPT, HC), jnp.float32),
        pltpu.VMEM((NPT, HC), jnp.float32),
        pltpu.VMEM((NPT, HC), jnp.float32),
        pltpu.VMEM((NPT, HC), jnp.float32),
        pltpu.VMEM((NPT, HC), jnp.float32),
        pltpu.VMEM((NPT, HC), jnp.float32),
        pltpu.VMEM((CHUNK,), jnp.int32),
        pltpu.VMEM((CHUNK,), jnp.int32),
        pltpu.VMEM((CHUNK,), jnp.int32),
        pltpu.VMEM((CHUNK, HC), jnp.float32),
        pltpu.VMEM((CHUNK, HC), jnp.float32),
        pltpu.SemaphoreType.DMA,
    ],
)(_prop_body)


def kernel(x, edge_index, W1, b1, W2, b2):
    h = _mlp(x, W1, b1, W2, b2)
    h_pad = jnp.pad(h, ((0, N_PAD - N), (0, 0)))
    h2 = jnp.concatenate([h_pad[:, :HC], h_pad[:, HC:]], axis=0)
    src = edge_index[0].astype(jnp.int32)
    dst = edge_index[1].astype(jnp.int32)
    z2, _ = _propagate(src, dst, h2)
    z = jnp.concatenate([z2[:N], z2[N_PAD:N_PAD + N]], axis=1)
    return _log_softmax(z)


# speedup vs baseline: 22.2484x; 3.8180x over previous
import functools

import jax
import jax.numpy as jnp
from jax import lax
from jax.experimental import pallas as pl
from jax.experimental.pallas import tpu as pltpu
from jax.experimental.pallas import tpu_sc as plsc

N = 10000
E = 320000
F_IN = 128
H = 64
C = 32
K = 10
ALPHA = 0.1

NC = 2
NS = 16
HC = C // 2
N_PAD = 10240
NPT = N_PAD // NS
CHUNK = 128
NCHUNK = 160
NBUF = 4
NROUNDS = NCHUNK // NBUF
EPT = NCHUNK * CHUNK
E_PAD = EPT * NS


def _mlp_body(x_ref, w1_ref, b1_ref, w2_ref, b2_ref, o_ref):
    h1 = jnp.dot(x_ref[...], w1_ref[...], preferred_element_type=jnp.float32)
    h1 = jnp.maximum(h1 + b1_ref[...], 0.0)
    o_ref[...] = (
        jnp.dot(h1, w2_ref[...], preferred_element_type=jnp.float32)
        + b2_ref[...]
    )


def _mlp(x, W1, b1, W2, b2):
    blk = 1000
    return pl.pallas_call(
        _mlp_body,
        grid=(N // blk,),
        in_specs=[
            pl.BlockSpec((blk, F_IN), lambda i: (i, 0)),
            pl.BlockSpec((F_IN, H), lambda i: (0, 0)),
            pl.BlockSpec((1, H), lambda i: (0, 0)),
            pl.BlockSpec((H, C), lambda i: (0, 0)),
            pl.BlockSpec((1, C), lambda i: (0, 0)),
        ],
        out_specs=pl.BlockSpec((blk, C), lambda i: (i, 0)),
        out_shape=jax.ShapeDtypeStruct((N, C), jnp.float32),
    )(x, W1, b1.reshape(1, H), W2, b2.reshape(1, C))


def _lsm_body(z_ref, o_ref):
    z = z_ref[...]
    m = jnp.max(z, axis=1, keepdims=True)
    e = jnp.exp(z - m)
    s = jnp.sum(e, axis=1, keepdims=True)
    o_ref[...] = z - m - jnp.log(s)


def _log_softmax(z):
    blk = 1000
    return pl.pallas_call(
        _lsm_body,
        grid=(N // blk,),
        in_specs=[pl.BlockSpec((blk, C), lambda i: (i, 0))],
        out_specs=pl.BlockSpec((blk, C), lambda i: (i, 0)),
        out_shape=jax.ShapeDtypeStruct((N, C), jnp.float32),
    )(z)


def _prop_body(srcx_hbm, dst_hbm, h2_hbm, z2_hbm, zt_hbm,
               acc_sh, h_l, z_l, zt_l, dv_l, s_l, zeros_l,
               src_all, dst_all, rows0, rows1, rows2, rows3,
               srci0, srci1, srci2, srci3, dsti_v,
               sem0, sem1, sem2, sem3):
    rows_v = (rows0, rows1, rows2, rows3)
    srci_v = (srci0, srci1, srci2, srci3)
    sems = (sem0, sem1, sem2, sem3)

    def load_idx(buf, table, ci):
        for j in range(CHUNK // 16):
            buf[pl.ds(j * 16, 16)] = table[ci, pl.ds(j * 16, 16)]
    c = lax.axis_index("c")
    s = lax.axis_index("s")
    nid0 = s * NPT
    g0 = c * N_PAD + nid0

    pltpu.sync_copy(h2_hbm.at[pl.ds(g0, NPT)], h_l)
    pltpu.sync_copy(
        srcx_hbm.at[pl.ds((c * NS + s) * NCHUNK, NCHUNK)], src_all)
    pltpu.sync_copy(dst_hbm.at[pl.ds(s * NCHUNK, NCHUNK)], dst_all)

    def initrow(r, _):
        zeros_l[r, :] = jnp.zeros((16,), jnp.float32)
        return None
    lax.fori_loop(0, NPT, initrow, None)

    def onesrow(r, _):
        rows0[r, :] = jnp.ones((16,), jnp.float32)
        return None
    lax.fori_loop(0, CHUNK, onesrow, None)

    pltpu.sync_copy(zeros_l, acc_sh.at[pl.ds(nid0, NPT)])
    plsc.subcore_barrier()

    def degchunk(ci, _):
        load_idx(dsti_v, dst_all, ci)
        pltpu.sync_copy(rows0, acc_sh.at[dsti_v], add=True)
        return None
    lax.fori_loop(0, NCHUNK, degchunk, None)
    plsc.subcore_barrier()

    pltpu.sync_copy(acc_sh.at[pl.ds(nid0, NPT)], s_l)

    def dinvrow(r, _):
        x = s_l[r, :] + 1.0
        i = plsc.bitcast(x, jnp.int32)
        i = 0x5F3759DF - lax.shift_right_arithmetic(i, 1)
        y = plsc.bitcast(i, jnp.float32)
        y = y * (1.5 - 0.5 * x * y * y)
        y = y * (1.5 - 0.5 * x * y * y)
        y = y * (1.5 - 0.5 * x * y * y)
        dv_l[r, :] = y
        z_l[r, :] = h_l[r, :]
        return None
    lax.fori_loop(0, NPT, dinvrow, None)
    pltpu.sync_copy(zeros_l, acc_sh.at[pl.ds(nid0, NPT)])

    def gather_start(ci, b):
        load_idx(srci_v[b], src_all, ci)
        pltpu.async_copy(zt_hbm.at[srci_v[b]], rows_v[b], sems[b])

    def gather_wait(b):
        pltpu.make_async_copy(
            zt_hbm.at[srci_v[b]], rows_v[b], sems[b]).wait()

    def scatter(ci, b):
        load_idx(dsti_v, dst_all, ci)
        pltpu.sync_copy(rows_v[b], acc_sh.at[dsti_v], add=True)

    def k_iter(k, _):
        def ztrow(r, _):
            zt_l[r, :] = z_l[r, :] * dv_l[r, :]
            return None
        lax.fori_loop(0, NPT, ztrow, None)
        pltpu.sync_copy(zt_l, zt_hbm.at[pl.ds(g0, NPT)])
        plsc.subcore_barrier()

        for b in range(NBUF):
            gather_start(b, b)

        def rnd(r, _):
            for b in range(NBUF):
                ci = r * NBUF + b
                gather_wait(b)
                scatter(ci, b)
                gather_start(ci + NBUF, b)
            return None
        lax.fori_loop(0, NROUNDS - 1, rnd, None)
        for b in range(NBUF):
            ci = (NROUNDS - 1) * NBUF + b
            gather_wait(b)
            scatter(ci, b)
        plsc.subcore_barrier()

        pltpu.sync_copy(acc_sh.at[pl.ds(nid0, NPT)], s_l)
        pltpu.sync_copy(zeros_l, acc_sh.at[pl.ds(nid0, NPT)])

        def comb(r, _):
            dv = dv_l[r, :]
            agg = dv * s_l[r, :] + dv * dv * z_l[r, :]
            z_l[r, :] = (1.0 - ALPHA) * agg + ALPHA * h_l[r, :]
            return None
        lax.fori_loop(0, NPT, comb, None)
        return None
    lax.fori_loop(0, K, k_iter, None)

    pltpu.sync_copy(z_l, z2_hbm.at[pl.ds(g0, NPT)])


_propagate = functools.partial(
    pl.kernel,
    out_type=(
        jax.ShapeDtypeStruct((2 * N_PAD, HC), jnp.float32),
        jax.ShapeDtypeStruct((2 * N_PAD, HC), jnp.float32),
    ),
    mesh=plsc.VectorSubcoreMesh(core_axis_name="c", subcore_axis_name="s"),
    compiler_params=pltpu.CompilerParams(
        needs_layout_passes=False, use_tc_tiling_on_sc=False
    ),
    scratch_types=[
        pltpu.VMEM_SHARED((N_PAD, HC), jnp.float32),
        pltpu.VMEM((NPT, HC), jnp.float32),
        pltpu.VMEM((NPT, HC), jnp.float32),
        pltpu.VMEM((NPT, HC), jnp.float32),
        pltpu.VMEM((NPT, HC), jnp.float32),
        pltpu.VMEM((NPT, HC), jnp.float32),
        pltpu.VMEM((NPT, HC), jnp.float32),
        pltpu.VMEM((NCHUNK, CHUNK), jnp.int32),
        pltpu.VMEM((NCHUNK, CHUNK), jnp.int32),
        pltpu.VMEM((CHUNK, HC), jnp.float32),
        pltpu.VMEM((CHUNK, HC), jnp.float32),
        pltpu.VMEM((CHUNK, HC), jnp.float32),
        pltpu.VMEM((CHUNK, HC), jnp.float32),
        pltpu.VMEM((CHUNK,), jnp.int32),
        pltpu.VMEM((CHUNK,), jnp.int32),
        pltpu.VMEM((CHUNK,), jnp.int32),
        pltpu.VMEM((CHUNK,), jnp.int32),
        pltpu.VMEM((CHUNK,), jnp.int32),
        pltpu.SemaphoreType.DMA,
        pltpu.SemaphoreType.DMA,
        pltpu.SemaphoreType.DMA,
        pltpu.SemaphoreType.DMA,
    ],
)(_prop_body)


def kernel(x, edge_index, W1, b1, W2, b2):
    h = _mlp(x, W1, b1, W2, b2)
    h_pad = jnp.pad(h, ((0, N_PAD - N), (0, 0)))
    h2 = jnp.concatenate([h_pad[:, :HC], h_pad[:, HC:]], axis=0)
    src = jnp.pad(edge_index[0].astype(jnp.int32), (0, E_PAD - E),
                  constant_values=N)
    dst = jnp.pad(edge_index[1].astype(jnp.int32), (0, E_PAD - E),
                  constant_values=N)
    srcx = jnp.concatenate([src, src + N_PAD]).reshape(-1, CHUNK)
    dst2 = dst.reshape(-1, CHUNK)
    z2, _ = _propagate(srcx, dst2, h2)
    z = jnp.concatenate([z2[:N], z2[N_PAD:N_PAD + N]], axis=1)
    return _log_softmax(z)


# speedup vs baseline: 22.8879x; 1.0287x over previous
import functools

import jax
import jax.numpy as jnp
from jax import lax
from jax.experimental import pallas as pl
from jax.experimental.pallas import tpu as pltpu
from jax.experimental.pallas import tpu_sc as plsc

N = 10000
E = 320000
F_IN = 128
H = 64
C = 32
K = 10
ALPHA = 0.1

NC = 2
NS = 16
HC = C // 2
N_PAD = 10240
NPT = N_PAD // NS
CHUNK = 128
NCHUNK = 160
NBUF = 8
LAG = 4
NROUNDS = NCHUNK // NBUF
EPT = NCHUNK * CHUNK
E_PAD = EPT * NS


def _mlp_body(x_ref, w1_ref, b1_ref, w2_ref, b2_ref, o_ref):
    h1 = jnp.dot(x_ref[...], w1_ref[...], preferred_element_type=jnp.float32)
    h1 = jnp.maximum(h1 + b1_ref[...], 0.0)
    o_ref[...] = (
        jnp.dot(h1, w2_ref[...], preferred_element_type=jnp.float32)
        + b2_ref[...]
    )


def _mlp(x, W1, b1, W2, b2):
    blk = 1000
    return pl.pallas_call(
        _mlp_body,
        grid=(N // blk,),
        in_specs=[
            pl.BlockSpec((blk, F_IN), lambda i: (i, 0)),
            pl.BlockSpec((F_IN, H), lambda i: (0, 0)),
            pl.BlockSpec((1, H), lambda i: (0, 0)),
            pl.BlockSpec((H, C), lambda i: (0, 0)),
            pl.BlockSpec((1, C), lambda i: (0, 0)),
        ],
        out_specs=pl.BlockSpec((blk, C), lambda i: (i, 0)),
        out_shape=jax.ShapeDtypeStruct((N, C), jnp.float32),
    )(x, W1, b1.reshape(1, H), W2, b2.reshape(1, C))


def _lsm_body(z_ref, o_ref):
    z = z_ref[...]
    m = jnp.max(z, axis=1, keepdims=True)
    e = jnp.exp(z - m)
    s = jnp.sum(e, axis=1, keepdims=True)
    o_ref[...] = z - m - jnp.log(s)


def _log_softmax(z):
    blk = 1000
    return pl.pallas_call(
        _lsm_body,
        grid=(N // blk,),
        in_specs=[pl.BlockSpec((blk, C), lambda i: (i, 0))],
        out_specs=pl.BlockSpec((blk, C), lambda i: (i, 0)),
        out_shape=jax.ShapeDtypeStruct((N, C), jnp.float32),
    )(z)


def _prop_body(srcx_hbm, dst_hbm, h2_hbm, z2_hbm, zt_hbm,
               acc_sh, h_l, z_l, zt_l, dv_l, s_l, zeros_l,
               src_all, dst_all, rows_v, srci_v, dsti_v, semg, sems):

    def load_idx(buf, table, ci):
        for j in range(CHUNK // 16):
            buf[pl.ds(j * 16, 16)] = table[ci, pl.ds(j * 16, 16)]
    c = lax.axis_index("c")
    s = lax.axis_index("s")
    nid0 = s * NPT
    g0 = c * N_PAD + nid0

    pltpu.sync_copy(h2_hbm.at[pl.ds(g0, NPT)], h_l)
    pltpu.sync_copy(
        srcx_hbm.at[pl.ds((c * NS + s) * NCHUNK, NCHUNK)], src_all)
    pltpu.sync_copy(dst_hbm.at[pl.ds(s * NCHUNK, NCHUNK)], dst_all)

    def initrow(r, _):
        zeros_l[r, :] = jnp.zeros((16,), jnp.float32)
        return None
    lax.fori_loop(0, NPT, initrow, None)

    def onesrow(r, _):
        rows_v[0][r, :] = jnp.ones((16,), jnp.float32)
        return None
    lax.fori_loop(0, CHUNK, onesrow, None)

    pltpu.sync_copy(zeros_l, acc_sh.at[pl.ds(nid0, NPT)])
    plsc.subcore_barrier()

    def degchunk(ci, _):
        load_idx(dsti_v[0], dst_all, ci)
        pltpu.sync_copy(rows_v[0], acc_sh.at[dsti_v[0]], add=True)
        return None
    lax.fori_loop(0, NCHUNK, degchunk, None)
    plsc.subcore_barrier()

    pltpu.sync_copy(acc_sh.at[pl.ds(nid0, NPT)], s_l)

    def dinvrow(r, _):
        x = s_l[r, :] + 1.0
        i = plsc.bitcast(x, jnp.int32)
        i = 0x5F3759DF - lax.shift_right_arithmetic(i, 1)
        y = plsc.bitcast(i, jnp.float32)
        y = y * (1.5 - 0.5 * x * y * y)
        y = y * (1.5 - 0.5 * x * y * y)
        y = y * (1.5 - 0.5 * x * y * y)
        dv_l[r, :] = y
        z_l[r, :] = h_l[r, :]
        return None
    lax.fori_loop(0, NPT, dinvrow, None)
    pltpu.sync_copy(zeros_l, acc_sh.at[pl.ds(nid0, NPT)])

    def g_start(ci, b):
        load_idx(srci_v[b], src_all, ci)
        pltpu.async_copy(zt_hbm.at[srci_v[b]], rows_v[b], semg[b])

    def g_wait(b):
        pltpu.make_async_copy(
            zt_hbm.at[srci_v[b]], rows_v[b], semg[b]).wait()

    def s_start(ci, b):
        load_idx(dsti_v[b], dst_all, ci)
        pltpu.async_copy(rows_v[b], acc_sh.at[dsti_v[b]], sems[b], add=True)

    def s_wait(b):
        pltpu.make_async_copy(
            rows_v[b], acc_sh.at[dsti_v[b]], sems[b]).wait()

    def k_iter(k, _):
        def ztrow(r, _):
            zt_l[r, :] = z_l[r, :] * dv_l[r, :]
            return None
        lax.fori_loop(0, NPT, ztrow, None)
        pltpu.sync_copy(zt_l, zt_hbm.at[pl.ds(g0, NPT)])
        plsc.subcore_barrier()

        for ci in range(LAG):
            g_start(ci, ci % NBUF)
        for ci in range(NBUF):
            b = ci % NBUF
            g_wait(b)
            s_start(ci, b)
            if ci >= LAG:
                s_wait((ci + LAG) % NBUF)
            g_start(ci + LAG, (ci + LAG) % NBUF)

        def rnd(r, _):
            for b in range(NBUF):
                ci = r * NBUF + b
                g_wait(b)
                s_start(ci, b)
                s_wait((b + LAG) % NBUF)
                g_start(ci + LAG, (b + LAG) % NBUF)
            return None
        lax.fori_loop(1, NROUNDS - 1, rnd, None)

        for ci in range((NROUNDS - 1) * NBUF, NCHUNK):
            b = ci % NBUF
            g_wait(b)
            s_start(ci, b)
            s_wait((ci + LAG) % NBUF)
            if ci + LAG < NCHUNK:
                g_start(ci + LAG, (ci + LAG) % NBUF)
        for ci in range(NCHUNK - LAG, NCHUNK):
            s_wait(ci % NBUF)
        plsc.subcore_barrier()

        pltpu.sync_copy(acc_sh.at[pl.ds(nid0, NPT)], s_l)
        pltpu.sync_copy(zeros_l, acc_sh.at[pl.ds(nid0, NPT)])

        def comb(r, _):
            dv = dv_l[r, :]
            agg = dv * s_l[r, :] + dv * dv * z_l[r, :]
            z_l[r, :] = (1.0 - ALPHA) * agg + ALPHA * h_l[r, :]
            return None
        lax.fori_loop(0, NPT, comb, None)
        return None
    lax.fori_loop(0, K, k_iter, None)

    pltpu.sync_copy(z_l, z2_hbm.at[pl.ds(g0, NPT)])


_propagate = functools.partial(
    pl.kernel,
    out_type=(
        jax.ShapeDtypeStruct((2 * N_PAD, HC), jnp.float32),
        jax.ShapeDtypeStruct((2 * N_PAD, HC), jnp.float32),
    ),
    mesh=plsc.VectorSubcoreMesh(core_axis_name="c", subcore_axis_name="s"),
    compiler_params=pltpu.CompilerParams(
        needs_layout_passes=False, use_tc_tiling_on_sc=False
    ),
    scratch_types=[
        pltpu.VMEM_SHARED((N_PAD, HC), jnp.float32),
        pltpu.VMEM((NPT, HC), jnp.float32),
        pltpu.VMEM((NPT, HC), jnp.float32),
        pltpu.VMEM((NPT, HC), jnp.float32),
        pltpu.VMEM((NPT, HC), jnp.float32),
        pltpu.VMEM((NPT, HC), jnp.float32),
        pltpu.VMEM((NPT, HC), jnp.float32),
        pltpu.VMEM((NCHUNK, CHUNK), jnp.int32),
        pltpu.VMEM((NCHUNK, CHUNK), jnp.int32),
        [pltpu.VMEM((CHUNK, HC), jnp.float32)] * NBUF,
        [pltpu.VMEM((CHUNK,), jnp.int32)] * NBUF,
        [pltpu.VMEM((CHUNK,), jnp.int32)] * NBUF,
        [pltpu.SemaphoreType.DMA] * NBUF,
        [pltpu.SemaphoreType.DMA] * NBUF,
    ],
)(_prop_body)


def kernel(x, edge_index, W1, b1, W2, b2):
    h = _mlp(x, W1, b1, W2, b2)
    h_pad = jnp.pad(h, ((0, N_PAD - N), (0, 0)))
    h2 = jnp.concatenate([h_pad[:, :HC], h_pad[:, HC:]], axis=0)
    src = jnp.pad(edge_index[0].astype(jnp.int32), (0, E_PAD - E),
                  constant_values=N)
    dst = jnp.pad(edge_index[1].astype(jnp.int32), (0, E_PAD - E),
                  constant_values=N)
    srcx = jnp.concatenate([src, src + N_PAD]).reshape(-1, CHUNK)
    dst2 = dst.reshape(-1, CHUNK)
    z2, _ = _propagate(srcx, dst2, h2)
    z = jnp.concatenate([z2[:N], z2[N_PAD:N_PAD + N]], axis=1)
    return _log_softmax(z)


# speedup vs baseline: 46.8458x; 2.0468x over previous
import functools

import jax
import jax.numpy as jnp
from jax import lax
from jax.experimental import pallas as pl
from jax.experimental.pallas import tpu as pltpu
from jax.experimental.pallas import tpu_sc as plsc

N = 10000
E = 320000
F_IN = 128
H = 64
C = 32
K = 10
ALPHA = 0.1

NC = 2
NS = 16
HC = C // 2
N_PAD = 10240
NPT = N_PAD // NS
CHUNK = 128
NCHUNK = 160
NBUF = 8
LAG = 4
NROUNDS = NCHUNK // NBUF
EPT = NCHUNK * CHUNK
E_PAD = EPT * NS


def _mlp_body(x_ref, w1_ref, b1_ref, w2_ref, b2_ref, o_ref):
    h1 = jnp.dot(x_ref[...], w1_ref[...], preferred_element_type=jnp.float32)
    h1 = jnp.maximum(h1 + b1_ref[...], 0.0)
    o_ref[...] = (
        jnp.dot(h1, w2_ref[...], preferred_element_type=jnp.float32)
        + b2_ref[...]
    )


def _mlp(x, W1, b1, W2, b2):
    blk = 1000
    return pl.pallas_call(
        _mlp_body,
        grid=(N // blk,),
        in_specs=[
            pl.BlockSpec((blk, F_IN), lambda i: (i, 0)),
            pl.BlockSpec((F_IN, H), lambda i: (0, 0)),
            pl.BlockSpec((1, H), lambda i: (0, 0)),
            pl.BlockSpec((H, C), lambda i: (0, 0)),
            pl.BlockSpec((1, C), lambda i: (0, 0)),
        ],
        out_specs=pl.BlockSpec((blk, C), lambda i: (i, 0)),
        out_shape=jax.ShapeDtypeStruct((N, C), jnp.float32),
    )(x, W1, b1.reshape(1, H), W2, b2.reshape(1, C))


def _lsm_body(z_ref, o_ref):
    z = z_ref[...]
    m = jnp.max(z, axis=1, keepdims=True)
    e = jnp.exp(z - m)
    s = jnp.sum(e, axis=1, keepdims=True)
    o_ref[...] = z - m - jnp.log(s)


def _log_softmax(z):
    blk = 1000
    return pl.pallas_call(
        _lsm_body,
        grid=(N // blk,),
        in_specs=[pl.BlockSpec((blk, C), lambda i: (i, 0))],
        out_specs=pl.BlockSpec((blk, C), lambda i: (i, 0)),
        out_shape=jax.ShapeDtypeStruct((N, C), jnp.float32),
    )(z)


def _prop_body(srcx_hbm, dst_hbm, h2_hbm, z2_hbm,
               acc_sh, zt_sh, h_l, z_l, dv_l, s_l, zeros_l,
               src_all, dst_all, rows_v, srci_v, dsti_v, semg, sems):

    def load_idx(buf, table, ci):
        for j in range(CHUNK // 16):
            buf[pl.ds(j * 16, 16)] = table[ci, pl.ds(j * 16, 16)]
    c = lax.axis_index("c")
    s = lax.axis_index("s")
    nid0 = s * NPT
    g0 = c * N_PAD + nid0

    pltpu.sync_copy(h2_hbm.at[pl.ds(g0, NPT)], h_l)
    pltpu.sync_copy(srcx_hbm.at[pl.ds(s * NCHUNK, NCHUNK)], src_all)
    pltpu.sync_copy(dst_hbm.at[pl.ds(s * NCHUNK, NCHUNK)], dst_all)

    def initrow(r, _):
        zeros_l[r, :] = jnp.zeros((16,), jnp.float32)
        return None
    lax.fori_loop(0, NPT, initrow, None)

    def onesrow(r, _):
        rows_v[0][r, :] = jnp.ones((16,), jnp.float32)
        return None
    lax.fori_loop(0, CHUNK, onesrow, None)

    pltpu.sync_copy(zeros_l, acc_sh.at[pl.ds(nid0, NPT)])
    plsc.subcore_barrier()

    def degchunk(ci, _):
        load_idx(dsti_v[0], dst_all, ci)
        pltpu.sync_copy(rows_v[0], acc_sh.at[dsti_v[0]], add=True)
        return None
    lax.fori_loop(0, NCHUNK, degchunk, None)
    plsc.subcore_barrier()

    pltpu.sync_copy(acc_sh.at[pl.ds(nid0, NPT)], s_l)

    def dinvrow(r, _):
        x = s_l[r, :] + 1.0
        i = plsc.bitcast(x, jnp.int32)
        i = 0x5F3759DF - lax.shift_right_arithmetic(i, 1)
        y = plsc.bitcast(i, jnp.float32)
        y = y * (1.5 - 0.5 * x * y * y)
        y = y * (1.5 - 0.5 * x * y * y)
        y = y * (1.5 - 0.5 * x * y * y)
        dv_l[r, :] = y
        z_l[r, :] = h_l[r, :]
        return None
    lax.fori_loop(0, NPT, dinvrow, None)
    pltpu.sync_copy(zeros_l, acc_sh.at[pl.ds(nid0, NPT)])

    def g_start(ci, b):
        load_idx(srci_v[b], src_all, ci)
        pltpu.async_copy(zt_sh.at[srci_v[b]], rows_v[b], semg[b])

    def g_wait(b):
        pltpu.make_async_copy(
            zt_sh.at[srci_v[b]], rows_v[b], semg[b]).wait()

    def s_start(ci, b):
        load_idx(dsti_v[b], dst_all, ci)
        pltpu.async_copy(rows_v[b], acc_sh.at[dsti_v[b]], sems[b], add=True)

    def s_wait(b):
        pltpu.make_async_copy(
            rows_v[b], acc_sh.at[dsti_v[b]], sems[b]).wait()

    def k_iter(k, _):
        def ztrow(r, _):
            s_l[r, :] = z_l[r, :] * dv_l[r, :]
            return None
        lax.fori_loop(0, NPT, ztrow, None)
        pltpu.sync_copy(s_l, zt_sh.at[pl.ds(nid0, NPT)])
        plsc.subcore_barrier()

        for ci in range(LAG):
            g_start(ci, ci % NBUF)
        for ci in range(NBUF):
            b = ci % NBUF
            g_wait(b)
            s_start(ci, b)
            if ci >= LAG:
                s_wait((ci + LAG) % NBUF)
            g_start(ci + LAG, (ci + LAG) % NBUF)

        def rnd(r, _):
            for b in range(NBUF):
                ci = r * NBUF + b
                g_wait(b)
                s_start(ci, b)
                s_wait((b + LAG) % NBUF)
                g_start(ci + LAG, (b + LAG) % NBUF)
            return None
        lax.fori_loop(1, NROUNDS - 1, rnd, None)

        for ci in range((NROUNDS - 1) * NBUF, NCHUNK):
            b = ci % NBUF
            g_wait(b)
            s_start(ci, b)
            s_wait((ci + LAG) % NBUF)
            if ci + LAG < NCHUNK:
                g_start(ci + LAG, (ci + LAG) % NBUF)
        for ci in range(NCHUNK - LAG, NCHUNK):
            s_wait(ci % NBUF)
        plsc.subcore_barrier()

        pltpu.sync_copy(acc_sh.at[pl.ds(nid0, NPT)], s_l)
        pltpu.sync_copy(zeros_l, acc_sh.at[pl.ds(nid0, NPT)])

        def comb(r, _):
            dv = dv_l[r, :]
            agg = dv * s_l[r, :] + dv * dv * z_l[r, :]
            z_l[r, :] = (1.0 - ALPHA) * agg + ALPHA * h_l[r, :]
            return None
        lax.fori_loop(0, NPT, comb, None)
        return None
    lax.fori_loop(0, K, k_iter, None)

    pltpu.sync_copy(z_l, z2_hbm.at[pl.ds(g0, NPT)])


_propagate = functools.partial(
    pl.kernel,
    out_type=jax.ShapeDtypeStruct((2 * N_PAD, HC), jnp.float32),
    mesh=plsc.VectorSubcoreMesh(core_axis_name="c", subcore_axis_name="s"),
    compiler_params=pltpu.CompilerParams(
        needs_layout_passes=False, use_tc_tiling_on_sc=False
    ),
    scratch_types=[
        pltpu.VMEM_SHARED((N_PAD, HC), jnp.float32),
        pltpu.VMEM_SHARED((N_PAD, HC), jnp.float32),
        pltpu.VMEM((NPT, HC), jnp.float32),
        pltpu.VMEM((NPT, HC), jnp.float32),
        pltpu.VMEM((NPT, HC), jnp.float32),
        pltpu.VMEM((NPT, HC), jnp.float32),
        pltpu.VMEM((NPT, HC), jnp.float32),
        pltpu.VMEM((NCHUNK, CHUNK), jnp.int32),
        pltpu.VMEM((NCHUNK, CHUNK), jnp.int32),
        [pltpu.VMEM((CHUNK, HC), jnp.float32)] * NBUF,
        [pltpu.VMEM((CHUNK,), jnp.int32)] * NBUF,
        [pltpu.VMEM((CHUNK,), jnp.int32)] * NBUF,
        [pltpu.SemaphoreType.DMA] * NBUF,
        [pltpu.SemaphoreType.DMA] * NBUF,
    ],
)(_prop_body)


def kernel(x, edge_index, W1, b1, W2, b2):
    h = _mlp(x, W1, b1, W2, b2)
    h_pad = jnp.pad(h, ((0, N_PAD - N), (0, 0)))
    h2 = jnp.concatenate([h_pad[:, :HC], h_pad[:, HC:]], axis=0)
    src2 = jnp.pad(edge_index[0].astype(jnp.int32), (0, E_PAD - E),
                   constant_values=N).reshape(-1, CHUNK)
    dst2 = jnp.pad(edge_index[1].astype(jnp.int32), (0, E_PAD - E),
                   constant_values=N).reshape(-1, CHUNK)
    z2 = _propagate(src2, dst2, h2)
    z = jnp.concatenate([z2[:N], z2[N_PAD:N_PAD + N]], axis=1)
    return _log_softmax(z)


# speedup vs baseline: 47.9345x; 1.0232x over previous
import functools

import jax
import jax.numpy as jnp
from jax import lax
from jax.experimental import pallas as pl
from jax.experimental.pallas import tpu as pltpu
from jax.experimental.pallas import tpu_sc as plsc

N = 10000
E = 320000
F_IN = 128
H = 64
C = 32
K = 10
ALPHA = 0.1

NC = 2
NS = 16
HC = C // 2
N_PAD = 10240
NPT = N_PAD // NS
CHUNK = 128
NCHUNK = 160
NBUF = 8
LAG = 4
NROUNDS = NCHUNK // NBUF
EPT = NCHUNK * CHUNK
E_PAD = EPT * NS


def _mlp_body(x_ref, w1_ref, b1_ref, w2_ref, b2_ref, o_ref):
    h1 = jnp.dot(x_ref[...], w1_ref[...], preferred_element_type=jnp.float32)
    h1 = jnp.maximum(h1 + b1_ref[...], 0.0)
    o_ref[...] = (
        jnp.dot(h1, w2_ref[...], preferred_element_type=jnp.float32)
        + b2_ref[...]
    )


def _mlp(x, W1, b1, W2, b2):
    blk = 1000
    return pl.pallas_call(
        _mlp_body,
        grid=(N // blk,),
        in_specs=[
            pl.BlockSpec((blk, F_IN), lambda i: (i, 0)),
            pl.BlockSpec((F_IN, H), lambda i: (0, 0)),
            pl.BlockSpec((1, H), lambda i: (0, 0)),
            pl.BlockSpec((H, C), lambda i: (0, 0)),
            pl.BlockSpec((1, C), lambda i: (0, 0)),
        ],
        out_specs=pl.BlockSpec((blk, C), lambda i: (i, 0)),
        out_shape=jax.ShapeDtypeStruct((N, C), jnp.float32),
    )(x, W1, b1.reshape(1, H), W2, b2.reshape(1, C))


def _lsm_body(z_ref, o_ref):
    z = z_ref[...]
    m = jnp.max(z, axis=1, keepdims=True)
    e = jnp.exp(z - m)
    s = jnp.sum(e, axis=1, keepdims=True)
    o_ref[...] = z - m - jnp.log(s)


def _log_softmax(z):
    blk = 1000
    return pl.pallas_call(
        _lsm_body,
        grid=(N // blk,),
        in_specs=[pl.BlockSpec((blk, C), lambda i: (i, 0))],
        out_specs=pl.BlockSpec((blk, C), lambda i: (i, 0)),
        out_shape=jax.ShapeDtypeStruct((N, C), jnp.float32),
    )(z)


def _prop_body(srcx_hbm, dst_hbm, h2_hbm, z2_hbm,
               acc_sh, zt_sh, h_l, z_l, dv_l, s_l, zeros_l,
               src_all, dst_all, rows_v, srci_v, dsti_v, semg, sems):

    def load_idx(buf, table, ci):
        for j in range(CHUNK // 16):
            buf[pl.ds(j * 16, 16)] = table[ci, pl.ds(j * 16, 16)]
    c = lax.axis_index("c")
    s = lax.axis_index("s")
    nid0 = s * NPT
    g0 = c * N_PAD + nid0

    pltpu.sync_copy(h2_hbm.at[pl.ds(g0, NPT)], h_l)
    pltpu.sync_copy(srcx_hbm.at[pl.ds(s * NCHUNK, NCHUNK)], src_all)
    pltpu.sync_copy(dst_hbm.at[pl.ds(s * NCHUNK, NCHUNK)], dst_all)

    def initrow(r, _):
        zeros_l[r, :] = jnp.zeros((16,), jnp.float32)
        return None
    lax.fori_loop(0, NPT, initrow, None)

    def onesrow(r, _):
        rows_v[0][r, :] = jnp.ones((16,), jnp.float32)
        return None
    lax.fori_loop(0, CHUNK, onesrow, None)

    pltpu.sync_copy(zeros_l, acc_sh.at[pl.ds(nid0, NPT)])
    plsc.subcore_barrier()

    def ds_start(ci, b):
        load_idx(dsti_v[b], dst_all, ci)
        pltpu.async_copy(rows_v[0], acc_sh.at[dsti_v[b]], sems[b], add=True)

    def ds_wait(b):
        pltpu.make_async_copy(
            rows_v[0], acc_sh.at[dsti_v[b]], sems[b]).wait()

    for ci in range(NBUF):
        ds_start(ci, ci)

    def degrnd(r, _):
        for b in range(NBUF):
            ds_wait(b)
            ds_start(r * NBUF + b, b)
        return None
    lax.fori_loop(1, NROUNDS, degrnd, None)
    for b in range(NBUF):
        ds_wait(b)
    plsc.subcore_barrier()

    pltpu.sync_copy(acc_sh.at[pl.ds(nid0, NPT)], s_l)

    def dinvrow(r, _):
        x = s_l[r, :] + 1.0
        i = plsc.bitcast(x, jnp.int32)
        i = 0x5F3759DF - lax.shift_right_arithmetic(i, 1)
        y = plsc.bitcast(i, jnp.float32)
        y = y * (1.5 - 0.5 * x * y * y)
        y = y * (1.5 - 0.5 * x * y * y)
        y = y * (1.5 - 0.5 * x * y * y)
        dv_l[r, :] = y
        hr = h_l[r, :]
        z_l[r, :] = hr
        s_l[r, :] = hr * y
        return None
    lax.fori_loop(0, NPT, dinvrow, None)
    pltpu.sync_copy(s_l, zt_sh.at[pl.ds(nid0, NPT)])
    pltpu.sync_copy(zeros_l, acc_sh.at[pl.ds(nid0, NPT)])

    def g_start(ci, b):
        load_idx(srci_v[b], src_all, ci)
        pltpu.async_copy(zt_sh.at[srci_v[b]], rows_v[b], semg[b])

    def g_wait(b):
        pltpu.make_async_copy(
            zt_sh.at[srci_v[b]], rows_v[b], semg[b]).wait()

    def s_start(ci, b):
        load_idx(dsti_v[b], dst_all, ci)
        pltpu.async_copy(rows_v[b], acc_sh.at[dsti_v[b]], sems[b], add=True)

    def s_wait(b):
        pltpu.make_async_copy(
            rows_v[b], acc_sh.at[dsti_v[b]], sems[b]).wait()

    def k_iter(k, _):
        plsc.subcore_barrier()

        for ci in range(LAG):
            g_start(ci, ci % NBUF)
        for ci in range(NBUF):
            b = ci % NBUF
            g_wait(b)
            s_start(ci, b)
            if ci >= LAG:
                s_wait((ci + LAG) % NBUF)
            g_start(ci + LAG, (ci + LAG) % NBUF)

        def rnd(r, _):
            for b in range(NBUF):
                ci = r * NBUF + b
                g_wait(b)
                s_start(ci, b)
                s_wait((b + LAG) % NBUF)
                g_start(ci + LAG, (b + LAG) % NBUF)
            return None
        lax.fori_loop(1, NROUNDS - 1, rnd, None)

        for ci in range((NROUNDS - 1) * NBUF, NCHUNK):
            b = ci % NBUF
            g_wait(b)
            s_start(ci, b)
            s_wait((ci + LAG) % NBUF)
            if ci + LAG < NCHUNK:
                g_start(ci + LAG, (ci + LAG) % NBUF)
        for ci in range(NCHUNK - LAG, NCHUNK):
            s_wait(ci % NBUF)
        plsc.subcore_barrier()

        pltpu.sync_copy(acc_sh.at[pl.ds(nid0, NPT)], s_l)
        pltpu.sync_copy(zeros_l, acc_sh.at[pl.ds(nid0, NPT)])

        def comb(r, _):
            dv = dv_l[r, :]
            agg = dv * s_l[r, :] + dv * dv * z_l[r, :]
            z = (1.0 - ALPHA) * agg + ALPHA * h_l[r, :]
            z_l[r, :] = z
            s_l[r, :] = z * dv
            return None
        lax.fori_loop(0, NPT, comb, None)
        pltpu.sync_copy(s_l, zt_sh.at[pl.ds(nid0, NPT)])
        return None
    lax.fori_loop(0, K, k_iter, None)

    pltpu.sync_copy(z_l, z2_hbm.at[pl.ds(g0, NPT)])


_propagate = functools.partial(
    pl.kernel,
    out_type=jax.ShapeDtypeStruct((2 * N_PAD, HC), jnp.float32),
    mesh=plsc.VectorSubcoreMesh(core_axis_name="c", subcore_axis_name="s"),
    compiler_params=pltpu.CompilerParams(
        needs_layout_passes=False, use_tc_tiling_on_sc=False
    ),
    scratch_types=[
        pltpu.VMEM_SHARED((N_PAD, HC), jnp.float32),
        pltpu.VMEM_SHARED((N_PAD, HC), jnp.float32),
        pltpu.VMEM((NPT, HC), jnp.float32),
        pltpu.VMEM((NPT, HC), jnp.float32),
        pltpu.VMEM((NPT, HC), jnp.float32),
        pltpu.VMEM((NPT, HC), jnp.float32),
        pltpu.VMEM((NPT, HC), jnp.float32),
        pltpu.VMEM((NCHUNK, CHUNK), jnp.int32),
        pltpu.VMEM((NCHUNK, CHUNK), jnp.int32),
        [pltpu.VMEM((CHUNK, HC), jnp.float32)] * NBUF,
        [pltpu.VMEM((CHUNK,), jnp.int32)] * NBUF,
        [pltpu.VMEM((CHUNK,), jnp.int32)] * NBUF,
        [pltpu.SemaphoreType.DMA] * NBUF,
        [pltpu.SemaphoreType.DMA] * NBUF,
    ],
)(_prop_body)


def kernel(x, edge_index, W1, b1, W2, b2):
    h = _mlp(x, W1, b1, W2, b2)
    h_pad = jnp.pad(h, ((0, N_PAD - N), (0, 0)))
    h2 = jnp.concatenate([h_pad[:, :HC], h_pad[:, HC:]], axis=0)
    src2 = jnp.pad(edge_index[0].astype(jnp.int32), (0, E_PAD - E),
                   constant_values=N).reshape(-1, CHUNK)
    dst2 = jnp.pad(edge_index[1].astype(jnp.int32), (0, E_PAD - E),
                   constant_values=N).reshape(-1, CHUNK)
    z2 = _propagate(src2, dst2, h2)
    z = jnp.concatenate([z2[:N], z2[N_PAD:N_PAD + N]], axis=1)
    return _log_softmax(z)


# speedup vs baseline: 49.4615x; 1.0319x over previous
import functools

import jax
import jax.numpy as jnp
from jax import lax
from jax.experimental import pallas as pl
from jax.experimental.pallas import tpu as pltpu
from jax.experimental.pallas import tpu_sc as plsc

N = 10000
E = 320000
F_IN = 128
H = 64
C = 32
K = 10
ALPHA = 0.1

NC = 2
NS = 16
HC = C // 2
N_PAD = 10240
NPT = N_PAD // NS
CHUNK = 128
NCHUNK = 160
NBUF = 8
LAG = 4
NROUNDS = NCHUNK // NBUF
EPT = NCHUNK * CHUNK
E_PAD = EPT * NS


def _mlp_body(x_ref, w1_ref, b1_ref, w2_ref, b2_ref, oa_ref, ob_ref):
    h1 = jnp.dot(x_ref[...], w1_ref[...], preferred_element_type=jnp.float32)
    h1 = jnp.maximum(h1 + b1_ref[...], 0.0)
    h = (
        jnp.dot(h1, w2_ref[...], preferred_element_type=jnp.float32)
        + b2_ref[...]
    )
    oa_ref[...] = h[:, :HC]
    ob_ref[...] = h[:, HC:]


def _mlp(x, W1, b1, W2, b2):
    blk = 1000
    return pl.pallas_call(
        _mlp_body,
        grid=(N // blk,),
        in_specs=[
            pl.BlockSpec((blk, F_IN), lambda i: (i, 0)),
            pl.BlockSpec((F_IN, H), lambda i: (0, 0)),
            pl.BlockSpec((1, H), lambda i: (0, 0)),
            pl.BlockSpec((H, C), lambda i: (0, 0)),
            pl.BlockSpec((1, C), lambda i: (0, 0)),
        ],
        out_specs=[
            pl.BlockSpec((blk, HC), lambda i: (i, 0)),
            pl.BlockSpec((blk, HC), lambda i: (i, 0)),
        ],
        out_shape=[
            jax.ShapeDtypeStruct((N_PAD, HC), jnp.float32),
            jax.ShapeDtypeStruct((N_PAD, HC), jnp.float32),
        ],
    )(x, W1, b1.reshape(1, H), W2, b2.reshape(1, C))


def _lsm_body(za_ref, zb_ref, o_ref):
    z = jnp.concatenate([za_ref[...], zb_ref[...]], axis=1)
    m = jnp.max(z, axis=1, keepdims=True)
    e = jnp.exp(z - m)
    s = jnp.sum(e, axis=1, keepdims=True)
    o_ref[...] = z - m - jnp.log(s)


def _log_softmax(z2):
    blk = 80
    return pl.pallas_call(
        _lsm_body,
        grid=(N // blk,),
        in_specs=[
            pl.BlockSpec((blk, HC), lambda i: (i, 0)),
            pl.BlockSpec((blk, HC), lambda i: (N_PAD // blk + i, 0)),
        ],
        out_specs=pl.BlockSpec((blk, C), lambda i: (i, 0)),
        out_shape=jax.ShapeDtypeStruct((N, C), jnp.float32),
    )(z2, z2)


def _prop_body(srcx_hbm, dst_hbm, h2a_hbm, h2b_hbm, z2_hbm,
               acc_sh, zt_sh, h_l, z_l, dv_l, s_l, zeros_l,
               src_all, dst_all, rows_v, srci_v, dsti_v, semg, sems):

    def load_idx(buf, table, ci):
        for j in range(CHUNK // 16):
            buf[pl.ds(j * 16, 16)] = table[ci, pl.ds(j * 16, 16)]
    c = lax.axis_index("c")
    s = lax.axis_index("s")
    nid0 = s * NPT
    g0 = c * N_PAD + nid0

    @pl.when(c == 0)
    def _():
        pltpu.sync_copy(h2a_hbm.at[pl.ds(nid0, NPT)], h_l)

    @pl.when(c == 1)
    def _():
        pltpu.sync_copy(h2b_hbm.at[pl.ds(nid0, NPT)], h_l)

    pltpu.sync_copy(srcx_hbm.at[pl.ds(s * NCHUNK, NCHUNK)], src_all)
    pltpu.sync_copy(dst_hbm.at[pl.ds(s * NCHUNK, NCHUNK)], dst_all)

    def initrow(r4, _):
        for u in range(4):
            zeros_l[r4 * 4 + u, :] = jnp.zeros((16,), jnp.float32)
        return None
    lax.fori_loop(0, NPT // 4, initrow, None)

    def onesrow(r4, _):
        for u in range(4):
            rows_v[0][r4 * 4 + u, :] = jnp.ones((16,), jnp.float32)
        return None
    lax.fori_loop(0, CHUNK // 4, onesrow, None)

    pltpu.sync_copy(zeros_l, acc_sh.at[pl.ds(nid0, NPT)])
    plsc.subcore_barrier()

    def ds_start(ci, b):
        pltpu.async_copy(rows_v[0], acc_sh.at[dst_all.at[ci, 0]], sems[b],
                         add=True)

    def ds_wait(b):
        pltpu.make_async_copy(
            rows_v[0], acc_sh.at[dst_all.at[0, 0]], sems[b]).wait()

    for ci in range(NBUF):
        ds_start(ci, ci)

    def degrnd(r, _):
        for b in range(NBUF):
            ds_wait(b)
            ds_start(r * NBUF + b, b)
        return None
    lax.fori_loop(1, NROUNDS, degrnd, None)
    for b in range(NBUF):
        ds_wait(b)
    plsc.subcore_barrier()

    pltpu.sync_copy(acc_sh.at[pl.ds(nid0, NPT)], s_l)

    def dinvrow(r4, _):
      for u in range(4):
        r = r4 * 4 + u
        x = s_l[r, :] + 1.0
        i = plsc.bitcast(x, jnp.int32)
        i = 0x5F3759DF - lax.shift_right_arithmetic(i, 1)
        y = plsc.bitcast(i, jnp.float32)
        y = y * (1.5 - 0.5 * x * y * y)
        y = y * (1.5 - 0.5 * x * y * y)
        y = y * (1.5 - 0.5 * x * y * y)
        dv_l[r, :] = y
        vb = jnp.full((16,), nid0 + r < N, dtype=jnp.bool_)
        hr = jnp.where(vb, h_l[r, :], 0.0)
        h_l[r, :] = hr
        z_l[r, :] = hr
        s_l[r, :] = hr * y
      return None
    lax.fori_loop(0, NPT // 4, dinvrow, None)
    pltpu.sync_copy(s_l, zt_sh.at[pl.ds(nid0, NPT)])
    pltpu.sync_copy(zeros_l, acc_sh.at[pl.ds(nid0, NPT)])

    def g_start(ci, b):
        pltpu.async_copy(zt_sh.at[src_all.at[ci]], rows_v[b], semg[b])

    def g_wait(b):
        pltpu.make_async_copy(
            zt_sh.at[src_all.at[0]], rows_v[b], semg[b]).wait()

    def s_start(ci, b):
        pltpu.async_copy(rows_v[b], acc_sh.at[dst_all.at[ci, 0]], sems[b],
                         add=True)

    def s_wait(b):
        pltpu.make_async_copy(
            rows_v[b], acc_sh.at[dst_all.at[0, 0]], sems[b]).wait()

    def k_iter(k, _):
        plsc.subcore_barrier()

        for ci in range(LAG):
            g_start(ci, ci % NBUF)
        for ci in range(NBUF):
            b = ci % NBUF
            g_wait(b)
            s_start(ci, b)
            if ci >= LAG:
                s_wait((ci + LAG) % NBUF)
            g_start(ci + LAG, (ci + LAG) % NBUF)

        def rnd(r, _):
            for b in range(NBUF):
                ci = r * NBUF + b
                g_wait(b)
                s_start(ci, b)
                s_wait((b + LAG) % NBUF)
                g_start(ci + LAG, (b + LAG) % NBUF)
            return None
        lax.fori_loop(1, NROUNDS - 1, rnd, None)

        for ci in range((NROUNDS - 1) * NBUF, NCHUNK):
            b = ci % NBUF
            g_wait(b)
            s_start(ci, b)
            s_wait((ci + LAG) % NBUF)
            if ci + LAG < NCHUNK:
                g_start(ci + LAG, (ci + LAG) % NBUF)
        for ci in range(NCHUNK - LAG, NCHUNK):
            s_wait(ci % NBUF)
        plsc.subcore_barrier()

        pltpu.sync_copy(acc_sh.at[pl.ds(nid0, NPT)], s_l)
        pltpu.sync_copy(zeros_l, acc_sh.at[pl.ds(nid0, NPT)])

        def comb(r4, _):
            for u in range(4):
                r = r4 * 4 + u
                dv = dv_l[r, :]
                agg = dv * s_l[r, :] + dv * dv * z_l[r, :]
                z = (1.0 - ALPHA) * agg + ALPHA * h_l[r, :]
                z_l[r, :] = z
                s_l[r, :] = z * dv
            return None
        lax.fori_loop(0, NPT // 4, comb, None)
        pltpu.sync_copy(s_l, zt_sh.at[pl.ds(nid0, NPT)])
        return None
    lax.fori_loop(0, K, k_iter, None)

    pltpu.sync_copy(z_l, z2_hbm.at[pl.ds(g0, NPT)])


_propagate = functools.partial(
    pl.kernel,
    out_type=jax.ShapeDtypeStruct((2 * N_PAD, HC), jnp.float32),
    mesh=plsc.VectorSubcoreMesh(core_axis_name="c", subcore_axis_name="s"),
    compiler_params=pltpu.CompilerParams(
        needs_layout_passes=False, use_tc_tiling_on_sc=False,
        disable_bounds_checks=True
    ),
    scratch_types=[
        pltpu.VMEM_SHARED((N_PAD, HC), jnp.float32),
        pltpu.VMEM_SHARED((N_PAD, HC), jnp.float32),
        pltpu.VMEM((NPT, HC), jnp.float32),
        pltpu.VMEM((NPT, HC), jnp.float32),
        pltpu.VMEM((NPT, HC), jnp.float32),
        pltpu.VMEM((NPT, HC), jnp.float32),
        pltpu.VMEM((NPT, HC), jnp.float32),
        pltpu.VMEM((NCHUNK, CHUNK), jnp.int32),
        pltpu.VMEM((NCHUNK, 1, CHUNK), jnp.int32),
        [pltpu.VMEM((CHUNK, HC), jnp.float32)] * NBUF,
        [pltpu.VMEM((CHUNK,), jnp.int32)] * NBUF,
        [pltpu.VMEM((CHUNK,), jnp.int32)] * NBUF,
        [pltpu.SemaphoreType.DMA] * NBUF,
        [pltpu.SemaphoreType.DMA] * NBUF,
    ],
)(_prop_body)


def kernel(x, edge_index, W1, b1, W2, b2):
    h2a, h2b = _mlp(x, W1, b1, W2, b2)
    src2 = jnp.pad(edge_index[0].astype(jnp.int32), (0, E_PAD - E),
                   constant_values=N).reshape(-1, CHUNK)
    dst2 = jnp.pad(edge_index[1].astype(jnp.int32), (0, E_PAD - E),
                   constant_values=N).reshape(-1, 1, CHUNK)
    z2 = _propagate(src2, dst2, h2a, h2b)
    return _log_softmax(z2)
